# baseline (speedup 1.0000x reference)
import functools

import jax
import jax.numpy as jnp
from jax import lax
from jax.experimental import pallas as pl
from jax.experimental.pallas import tpu as pltpu
from jax.experimental.pallas import tpu_sc as plsc

N_NODES = 10000
N_EDGES = 320000
D = 128
DH = D // 2
N_GRAPHS = 64
BN_EPS = 1e-5

NUM_CORES = 2
NUM_SUBCORES = 16
NUM_WORKERS = NUM_CORES * NUM_SUBCORES
E_PER_TILE = N_EDGES // NUM_WORKERS
CHUNK = 100
N_CHUNKS = E_PER_TILE // CHUNK
NBUF = 5
N_ITER = N_CHUNKS // NBUF
N_PAD = 10240
ROWS_PER_TILE = N_PAD // NUM_SUBCORES

_sc_mesh = plsc.VectorSubcoreMesh(
    core_axis_name="c", subcore_axis_name="s",
    num_cores=NUM_CORES, num_subcores=NUM_SUBCORES)


@functools.partial(
    pl.kernel,
    out_type=jax.ShapeDtypeStruct((NUM_CORES, 2, N_PAD, DH), jnp.float32),
    mesh=_sc_mesh,
    scratch_types=[
        pltpu.VMEM((N_CHUNKS, CHUNK), jnp.int32),
        pltpu.VMEM((N_CHUNKS, CHUNK), jnp.int32),
        pltpu.VMEM((CHUNK, DH), jnp.float32),
        pltpu.VMEM((CHUNK, DH), jnp.float32),
        pltpu.VMEM((CHUNK, DH), jnp.float32),
        pltpu.VMEM((CHUNK, DH), jnp.float32),
        pltpu.VMEM((CHUNK, DH), jnp.float32),
        pltpu.VMEM((ROWS_PER_TILE // 2, DH), jnp.float32),
        pltpu.VMEM_SHARED((N_PAD, DH), jnp.float32),
        pltpu.SemaphoreType.DMA,
        pltpu.SemaphoreType.DMA,
        pltpu.SemaphoreType.DMA,
        pltpu.SemaphoreType.DMA,
        pltpu.SemaphoreType.DMA,
    ],
    compiler_params=pltpu.CompilerParams(use_tc_tiling_on_sc=False),
)
def _segment_sum_sc(hlo_hbm, hhi_hbm, col_hbm, row_hbm, out_hbm,
                    col_b, row_b, g0, g1, g2, g3, g4, zbuf, acc,
                    s0, s1, s2, s3, s4):
    bufs = (g0, g1, g2, g3, g4)
    sems = (s0, s1, s2, s3, s4)
    cid = lax.axis_index("c")
    sid = lax.axis_index("s")
    wid = sid * NUM_CORES + cid

    pltpu.sync_copy(col_hbm.at[wid], col_b)
    pltpu.sync_copy(row_hbm.at[wid], row_b)

    zv = jnp.zeros((16,), jnp.float32)

    def _zrow(r, carry):
        for c in range(DH // 16):
            zbuf[r, pl.ds(c * 16, 16)] = zv
        return carry

    lax.fori_loop(0, ROWS_PER_TILE // 2, _zrow, 0)

    for p, h_hbm in enumerate((hlo_hbm, hhi_hbm)):
        half = ROWS_PER_TILE // 2
        pltpu.sync_copy(zbuf, acc.at[pl.ds(sid * ROWS_PER_TILE, half)])
        pltpu.sync_copy(zbuf, acc.at[pl.ds(sid * ROWS_PER_TILE + half, half)])
        plsc.subcore_barrier()

        for b in range(NBUF):
            pltpu.async_copy(h_hbm.at[col_b.at[b]], bufs[b], sems[b])

        def _iter(i, carry):
            for b in range(NBUF):
                k = i * NBUF + b
                pltpu.make_async_copy(
                    h_hbm.at[col_b.at[0]], bufs[b], sems[b]).wait()
                pltpu.sync_copy(bufs[b], acc.at[row_b.at[k]], add=True)

                @pl.when(i < N_ITER - 1)
                def _fire():
                    pltpu.async_copy(
                        h_hbm.at[col_b.at[k + NBUF]], bufs[b], sems[b])
            return carry

        lax.fori_loop(0, N_ITER, _iter, 0)
        plsc.subcore_barrier()

        pltpu.sync_copy(
            acc.at[pl.ds(sid * ROWS_PER_TILE, ROWS_PER_TILE)],
            out_hbm.at[cid, p, pl.ds(sid * ROWS_PER_TILE, ROWS_PER_TILE)])


def _bn_relu(t, g, b):
    mean = jnp.mean(t, axis=0, keepdims=True)
    var = jnp.mean((t - mean) ** 2, axis=0, keepdims=True)
    return jnp.maximum(g * (t - mean) * lax.rsqrt(var + BN_EPS) + b, 0.0)


def _layer_tc_body(eps_ref, part_ref, h_ref, w0_ref, b0_ref, g0_ref, be0_ref,
                   w1_ref, b1_ref, g1_ref, be1_ref, out_ref):
    h = jnp.concatenate([h_ref[0], h_ref[1]], axis=1)
    seg = jnp.concatenate(
        [part_ref[0, 0, :N_NODES] + part_ref[1, 0, :N_NODES],
         part_ref[0, 1, :N_NODES] + part_ref[1, 1, :N_NODES]], axis=1)
    pooled = seg + (1.0 + eps_ref[0]) * h
    t = lax.dot_general(pooled, w0_ref[...], (((1,), (1,)), ((), ())),
                        preferred_element_type=jnp.float32) + b0_ref[...]
    t = _bn_relu(t, g0_ref[...], be0_ref[...])
    t = lax.dot_general(t, w1_ref[...], (((1,), (1,)), ((), ())),
                        preferred_element_type=jnp.float32) + b1_ref[...]
    y = _bn_relu(t, g1_ref[...], be1_ref[...])
    out_ref[0] = y[:, :DH]
    out_ref[1] = y[:, DH:]


_layer_tc = pl.pallas_call(
    _layer_tc_body,
    out_shape=jax.ShapeDtypeStruct((2, N_NODES, DH), jnp.float32),
    in_specs=[pl.BlockSpec(memory_space=pltpu.SMEM)]
    + [pl.BlockSpec(memory_space=pltpu.VMEM)] * 10,
)


def _pool_tc_body(batch_ref, h0, h1, h2, h3, h4, w0, w1, w2, w3, w4,
                  b0, b1, b2, b3, b4, out_ref):
    hs = (h0, h1, h2, h3, h4)
    ws = (w0, w1, w2, w3, w4)
    bs = (b0, b1, b2, b3, b4)
    gids = lax.broadcasted_iota(jnp.int32, (N_GRAPHS, N_NODES), 0)
    sel = (gids == batch_ref[...]).astype(jnp.float32)
    counts = jnp.maximum(jnp.sum(sel, axis=1, keepdims=True), 1.0)
    score = jnp.zeros((N_GRAPHS, D), jnp.float32)
    for l in range(5):
        h = jnp.concatenate([hs[l][0], hs[l][1]], axis=1)
        pooled = lax.dot_general(sel, h, (((1,), (0,)), ((), ())),
                                 preferred_element_type=jnp.float32) / counts
        score = score + lax.dot_general(
            pooled, ws[l][...], (((1,), (1,)), ((), ())),
            preferred_element_type=jnp.float32) + bs[l][...]
    out_ref[...] = score


_pool_tc = pl.pallas_call(
    _pool_tc_body,
    out_shape=jax.ShapeDtypeStruct((N_GRAPHS, D), jnp.float32),
    in_specs=[pl.BlockSpec(memory_space=pltpu.VMEM)] * 16,
)


def kernel(x, params, edge_index, batch):
    row = edge_index[0].reshape(NUM_WORKERS, N_CHUNKS, CHUNK)
    col = edge_index[1].reshape(NUM_WORKERS, N_CHUNKS, CHUNK)
    eps = params["eps"]
    h = jnp.stack([x[:, :DH], x[:, DH:]], axis=0)
    hidden = [h]
    for layer in range(4):
        mlp = params["mlp%d" % layer]
        parts = _segment_sum_sc(h[0], h[1], col, row)
        h = _layer_tc(
            eps[layer].reshape(1), parts, h,
            mlp["W0"], mlp["b0"].reshape(1, D),
            mlp["bn_g0"].reshape(1, D), mlp["bn_b0"].reshape(1, D),
            mlp["W1"], mlp["b1"].reshape(1, D),
            params["bn_g%d" % layer].reshape(1, D),
            params["bn_b%d" % layer].reshape(1, D))
        hidden.append(h)
    batch2d = batch.reshape(1, N_NODES)
    pred_ws = [params["pred%d_W" % l] for l in range(5)]
    pred_bs = [params["pred%d_b" % l].reshape(1, D) for l in range(5)]
    return _pool_tc(batch2d, *hidden, *pred_ws, *pred_bs)


# speedup vs baseline: 1.0003x; 1.0003x over previous
import functools

import jax
import jax.numpy as jnp
from jax import lax
from jax.experimental import pallas as pl
from jax.experimental.pallas import tpu as pltpu
from jax.experimental.pallas import tpu_sc as plsc

N_NODES = 10000
N_EDGES = 320000
D = 128
DH = D // 2
N_GRAPHS = 64
BN_EPS = 1e-5

NUM_CORES = 2
NUM_SUBCORES = 16
NUM_WORKERS = NUM_CORES * NUM_SUBCORES
E_PER_TILE = N_EDGES // NUM_WORKERS
CHUNK = 100
N_CHUNKS = E_PER_TILE // CHUNK
NBUF = 5
N_ITER = N_CHUNKS // NBUF
N_PAD = 10240
ROWS_PER_TILE = N_PAD // NUM_SUBCORES

_sc_mesh = plsc.VectorSubcoreMesh(
    core_axis_name="c", subcore_axis_name="s",
    num_cores=NUM_CORES, num_subcores=NUM_SUBCORES)


@functools.partial(
    pl.kernel,
    out_type=jax.ShapeDtypeStruct((NUM_CORES, 2, N_PAD, DH), jnp.float32),
    mesh=_sc_mesh,
    scratch_types=[
        pltpu.VMEM((N_CHUNKS, CHUNK), jnp.int32),
        pltpu.VMEM((N_CHUNKS, CHUNK), jnp.int32),
        pltpu.VMEM((CHUNK, DH), jnp.float32),
        pltpu.VMEM((CHUNK, DH), jnp.float32),
        pltpu.VMEM((CHUNK, DH), jnp.float32),
        pltpu.VMEM((CHUNK, DH), jnp.float32),
        pltpu.VMEM((CHUNK, DH), jnp.float32),
        pltpu.VMEM((ROWS_PER_TILE // 2, DH), jnp.float32),
        pltpu.VMEM_SHARED((N_PAD, DH), jnp.float32),
        pltpu.SemaphoreType.DMA,
        pltpu.SemaphoreType.DMA,
        pltpu.SemaphoreType.DMA,
        pltpu.SemaphoreType.DMA,
        pltpu.SemaphoreType.DMA,
    ],
    compiler_params=pltpu.CompilerParams(use_tc_tiling_on_sc=False, skip_device_barrier=True),
)
def _segment_sum_sc(hlo_hbm, hhi_hbm, col_hbm, row_hbm, out_hbm,
                    col_b, row_b, g0, g1, g2, g3, g4, zbuf, acc,
                    s0, s1, s2, s3, s4):
    bufs = (g0, g1, g2, g3, g4)
    sems = (s0, s1, s2, s3, s4)
    cid = lax.axis_index("c")
    sid = lax.axis_index("s")
    wid = sid * NUM_CORES + cid

    pltpu.sync_copy(col_hbm.at[wid], col_b)
    pltpu.sync_copy(row_hbm.at[wid], row_b)

    zv = jnp.zeros((16,), jnp.float32)

    def _zrow(r, carry):
        for c in range(DH // 16):
            zbuf[r, pl.ds(c * 16, 16)] = zv
        return carry

    lax.fori_loop(0, ROWS_PER_TILE // 2, _zrow, 0)

    for p, h_hbm in enumerate((hlo_hbm, hhi_hbm)):
        half = ROWS_PER_TILE // 2
        pltpu.sync_copy(zbuf, acc.at[pl.ds(sid * ROWS_PER_TILE, half)])
        pltpu.sync_copy(zbuf, acc.at[pl.ds(sid * ROWS_PER_TILE + half, half)])
        plsc.subcore_barrier()

        for b in range(NBUF):
            pltpu.async_copy(h_hbm.at[col_b.at[b]], bufs[b], sems[b])

        def _iter(i, carry):
            for b in range(NBUF):
                k = i * NBUF + b
                pltpu.make_async_copy(
                    h_hbm.at[col_b.at[0]], bufs[b], sems[b]).wait()
                pltpu.sync_copy(bufs[b], acc.at[row_b.at[k]], add=True)

                @pl.when(i < N_ITER - 1)
                def _fire():
                    pltpu.async_copy(
                        h_hbm.at[col_b.at[k + NBUF]], bufs[b], sems[b])
            return carry

        lax.fori_loop(0, N_ITER, _iter, 0)
        plsc.subcore_barrier()

        pltpu.sync_copy(
            acc.at[pl.ds(sid * ROWS_PER_TILE, ROWS_PER_TILE)],
            out_hbm.at[cid, p, pl.ds(sid * ROWS_PER_TILE, ROWS_PER_TILE)])


def _bn_relu(t, g, b):
    mean = jnp.mean(t, axis=0, keepdims=True)
    var = jnp.mean((t - mean) ** 2, axis=0, keepdims=True)
    return jnp.maximum(g * (t - mean) * lax.rsqrt(var + BN_EPS) + b, 0.0)


def _layer_tc_body(eps_ref, part_ref, h_ref, w0_ref, b0_ref, g0_ref, be0_ref,
                   w1_ref, b1_ref, g1_ref, be1_ref, out_ref):
    h = jnp.concatenate([h_ref[0], h_ref[1]], axis=1)
    seg = jnp.concatenate(
        [part_ref[0, 0, :N_NODES] + part_ref[1, 0, :N_NODES],
         part_ref[0, 1, :N_NODES] + part_ref[1, 1, :N_NODES]], axis=1)
    pooled = seg + (1.0 + eps_ref[0]) * h
    t = lax.dot_general(pooled, w0_ref[...], (((1,), (1,)), ((), ())),
                        preferred_element_type=jnp.float32) + b0_ref[...]
    t = _bn_relu(t, g0_ref[...], be0_ref[...])
    t = lax.dot_general(t, w1_ref[...], (((1,), (1,)), ((), ())),
                        preferred_element_type=jnp.float32) + b1_ref[...]
    y = _bn_relu(t, g1_ref[...], be1_ref[...])
    out_ref[0] = y[:, :DH]
    out_ref[1] = y[:, DH:]


_layer_tc = pl.pallas_call(
    _layer_tc_body,
    out_shape=jax.ShapeDtypeStruct((2, N_NODES, DH), jnp.float32),
    in_specs=[pl.BlockSpec(memory_space=pltpu.SMEM)]
    + [pl.BlockSpec(memory_space=pltpu.VMEM)] * 10,
)


def _pool_tc_body(batch_ref, h0, h1, h2, h3, h4, w0, w1, w2, w3, w4,
                  b0, b1, b2, b3, b4, out_ref):
    hs = (h0, h1, h2, h3, h4)
    ws = (w0, w1, w2, w3, w4)
    bs = (b0, b1, b2, b3, b4)
    gids = lax.broadcasted_iota(jnp.int32, (N_GRAPHS, N_NODES), 0)
    sel = (gids == batch_ref[...]).astype(jnp.float32)
    counts = jnp.maximum(jnp.sum(sel, axis=1, keepdims=True), 1.0)
    score = jnp.zeros((N_GRAPHS, D), jnp.float32)
    for l in range(5):
        h = jnp.concatenate([hs[l][0], hs[l][1]], axis=1)
        pooled = lax.dot_general(sel, h, (((1,), (0,)), ((), ())),
                                 preferred_element_type=jnp.float32) / counts
        score = score + lax.dot_general(
            pooled, ws[l][...], (((1,), (1,)), ((), ())),
            preferred_element_type=jnp.float32) + bs[l][...]
    out_ref[...] = score


_pool_tc = pl.pallas_call(
    _pool_tc_body,
    out_shape=jax.ShapeDtypeStruct((N_GRAPHS, D), jnp.float32),
    in_specs=[pl.BlockSpec(memory_space=pltpu.VMEM)] * 16,
)


def kernel(x, params, edge_index, batch):
    row = edge_index[0].reshape(NUM_WORKERS, N_CHUNKS, CHUNK)
    col = edge_index[1].reshape(NUM_WORKERS, N_CHUNKS, CHUNK)
    eps = params["eps"]
    h = jnp.stack([x[:, :DH], x[:, DH:]], axis=0)
    hidden = [h]
    for layer in range(4):
        mlp = params["mlp%d" % layer]
        parts = _segment_sum_sc(h[0], h[1], col, row)
        h = _layer_tc(
            eps[layer].reshape(1), parts, h,
            mlp["W0"], mlp["b0"].reshape(1, D),
            mlp["bn_g0"].reshape(1, D), mlp["bn_b0"].reshape(1, D),
            mlp["W1"], mlp["b1"].reshape(1, D),
            params["bn_g%d" % layer].reshape(1, D),
            params["bn_b%d" % layer].reshape(1, D))
        hidden.append(h)
    batch2d = batch.reshape(1, N_NODES)
    pred_ws = [params["pred%d_W" % l] for l in range(5)]
    pred_bs = [params["pred%d_b" % l].reshape(1, D) for l in range(5)]
    return _pool_tc(batch2d, *hidden, *pred_ws, *pred_bs)


# speedup vs baseline: 1.0274x; 1.0271x over previous
import functools

import jax
import jax.numpy as jnp
from jax import lax
from jax.experimental import pallas as pl
from jax.experimental.pallas import tpu as pltpu
from jax.experimental.pallas import tpu_sc as plsc

N_NODES = 10000
N_EDGES = 320000
D = 128
DH = D // 2
N_GRAPHS = 64
BN_EPS = 1e-5

NUM_CORES = 2
NUM_SUBCORES = 16
NUM_WORKERS = NUM_CORES * NUM_SUBCORES
E_PER_TILE = N_EDGES // NUM_WORKERS
CHUNK = 100
N_CHUNKS = E_PER_TILE // CHUNK
NBUF = 5
N_ITER = N_CHUNKS // NBUF
N_PAD = 10240
ROWS_PER_TILE = N_PAD // NUM_SUBCORES

_sc_mesh = plsc.VectorSubcoreMesh(
    core_axis_name="c", subcore_axis_name="s",
    num_cores=NUM_CORES, num_subcores=NUM_SUBCORES)


@functools.partial(
    pl.kernel,
    out_type=jax.ShapeDtypeStruct((NUM_CORES, 2, N_PAD, DH), jnp.float32),
    mesh=_sc_mesh,
    scratch_types=[
        pltpu.VMEM((N_CHUNKS, CHUNK), jnp.int32),
        pltpu.VMEM((N_CHUNKS, CHUNK), jnp.int32),
        pltpu.VMEM((CHUNK, DH), jnp.float32),
        pltpu.VMEM((CHUNK, DH), jnp.float32),
        pltpu.VMEM((CHUNK, DH), jnp.float32),
        pltpu.VMEM((CHUNK, DH), jnp.float32),
        pltpu.VMEM((CHUNK, DH), jnp.float32),
        pltpu.VMEM((ROWS_PER_TILE // 2, DH), jnp.float32),
        pltpu.VMEM_SHARED((N_PAD, DH), jnp.float32),
        pltpu.SemaphoreType.DMA,
        pltpu.SemaphoreType.DMA,
        pltpu.SemaphoreType.DMA,
        pltpu.SemaphoreType.DMA,
        pltpu.SemaphoreType.DMA,
    ],
    compiler_params=pltpu.CompilerParams(use_tc_tiling_on_sc=False, skip_device_barrier=True),
)
def _segment_sum_sc(hlo_hbm, hhi_hbm, col_hbm, row_hbm, out_hbm,
                    col_b, row_b, g0, g1, g2, g3, g4, zbuf, acc,
                    s0, s1, s2, s3, s4):
    bufs = (g0, g1, g2, g3, g4)
    sems = (s0, s1, s2, s3, s4)
    cid = lax.axis_index("c")
    sid = lax.axis_index("s")
    wid = sid * NUM_CORES + cid

    pltpu.sync_copy(col_hbm.at[wid], col_b)
    pltpu.sync_copy(row_hbm.at[wid], row_b)

    zv = jnp.zeros((16,), jnp.float32)

    def _zrow(r, carry):
        for c in range(DH // 16):
            zbuf[r, pl.ds(c * 16, 16)] = zv
        return carry

    lax.fori_loop(0, ROWS_PER_TILE // 2, _zrow, 0)

    for p, h_hbm in enumerate((hlo_hbm, hhi_hbm)):
        half = ROWS_PER_TILE // 2
        pltpu.sync_copy(zbuf, acc.at[pl.ds(sid * ROWS_PER_TILE, half)])
        pltpu.sync_copy(zbuf, acc.at[pl.ds(sid * ROWS_PER_TILE + half, half)])
        plsc.subcore_barrier()

        for b in range(NBUF):
            pltpu.async_copy(h_hbm.at[col_b.at[b]], bufs[b], sems[b])

        def _iter(i, carry):
            for b in range(NBUF):
                k = i * NBUF + b
                pltpu.make_async_copy(
                    h_hbm.at[col_b.at[0]], bufs[b], sems[b]).wait()
                pltpu.sync_copy(bufs[b], acc.at[row_b.at[k]], add=True)

                @pl.when(i < N_ITER - 1)
                def _fire():
                    pltpu.async_copy(
                        h_hbm.at[col_b.at[k + NBUF]], bufs[b], sems[b])
            return carry

        lax.fori_loop(0, N_ITER, _iter, 0)
        plsc.subcore_barrier()

        pltpu.sync_copy(
            acc.at[pl.ds(sid * ROWS_PER_TILE, ROWS_PER_TILE)],
            out_hbm.at[cid, p, pl.ds(sid * ROWS_PER_TILE, ROWS_PER_TILE)])


def _bn_relu(t, g, b):
    mean = jnp.mean(t, axis=0, keepdims=True)
    var = jnp.mean((t - mean) ** 2, axis=0, keepdims=True)
    return jnp.maximum(g * (t - mean) * lax.rsqrt(var + BN_EPS) + b, 0.0)


def _graph_pool(batch2d, h, pw, pb):
    gids = lax.broadcasted_iota(jnp.int32, (N_GRAPHS, N_NODES), 0)
    sel = (gids == batch2d).astype(jnp.float32)
    counts = jnp.maximum(jnp.sum(sel, axis=1, keepdims=True), 1.0)
    pooled = lax.dot_general(sel, h, (((1,), (0,)), ((), ())),
                             preferred_element_type=jnp.float32) / counts
    return lax.dot_general(pooled, pw, (((1,), (1,)), ((), ())),
                           preferred_element_type=jnp.float32) + pb


def _make_layer_tc(last):
    n_extra = 4 if last else 2

    def _layer_tc_body(eps_ref, part_ref, h_ref, w0_ref, b0_ref, g0_ref,
                       be0_ref, w1_ref, b1_ref, g1_ref, be1_ref, batch_ref,
                       pw_ref, pb_ref, *rest):
        if last:
            pw2_ref, pb2_ref, out_ref, score_ref = rest
        else:
            out_ref, score_ref = rest
        h = jnp.concatenate([h_ref[0], h_ref[1]], axis=1)
        seg = jnp.concatenate(
            [part_ref[0, 0, :N_NODES] + part_ref[1, 0, :N_NODES],
             part_ref[0, 1, :N_NODES] + part_ref[1, 1, :N_NODES]], axis=1)
        pooled = seg + (1.0 + eps_ref[0]) * h
        t = lax.dot_general(pooled, w0_ref[...], (((1,), (1,)), ((), ())),
                            preferred_element_type=jnp.float32) + b0_ref[...]
        t = _bn_relu(t, g0_ref[...], be0_ref[...])
        t = lax.dot_general(t, w1_ref[...], (((1,), (1,)), ((), ())),
                            preferred_element_type=jnp.float32) + b1_ref[...]
        y = _bn_relu(t, g1_ref[...], be1_ref[...])
        out_ref[0] = y[:, :DH]
        out_ref[1] = y[:, DH:]
        score = _graph_pool(batch_ref[...], h, pw_ref[...], pb_ref[...])
        if last:
            score = score + _graph_pool(batch_ref[...], y, pw2_ref[...],
                                        pb2_ref[...])
        score_ref[...] = score

    return pl.pallas_call(
        _layer_tc_body,
        out_shape=(jax.ShapeDtypeStruct((2, N_NODES, DH), jnp.float32),
                   jax.ShapeDtypeStruct((N_GRAPHS, D), jnp.float32)),
        in_specs=[pl.BlockSpec(memory_space=pltpu.SMEM)]
        + [pl.BlockSpec(memory_space=pltpu.VMEM)] * (11 + n_extra),
    )


_layer_tc = _make_layer_tc(False)
_layer_tc_last = _make_layer_tc(True)


def kernel(x, params, edge_index, batch):
    row = edge_index[0].reshape(NUM_WORKERS, N_CHUNKS, CHUNK)
    col = edge_index[1].reshape(NUM_WORKERS, N_CHUNKS, CHUNK)
    eps = params["eps"]
    h = jnp.stack([x[:, :DH], x[:, DH:]], axis=0)
    batch2d = batch.reshape(1, N_NODES)
    contribs = []
    for layer in range(4):
        mlp = params["mlp%d" % layer]
        parts = _segment_sum_sc(h[0], h[1], col, row)
        args = [
            eps[layer].reshape(1), parts, h,
            mlp["W0"], mlp["b0"].reshape(1, D),
            mlp["bn_g0"].reshape(1, D), mlp["bn_b0"].reshape(1, D),
            mlp["W1"], mlp["b1"].reshape(1, D),
            params["bn_g%d" % layer].reshape(1, D),
            params["bn_b%d" % layer].reshape(1, D),
            batch2d,
            params["pred%d_W" % layer], params["pred%d_b" % layer].reshape(1, D),
        ]
        if layer < 3:
            h, score = _layer_tc(*args)
        else:
            args += [params["pred4_W"], params["pred4_b"].reshape(1, D)]
            h, score = _layer_tc_last(*args)
        contribs.append(score)
    return contribs[0] + contribs[1] + contribs[2] + contribs[3]


# speedup vs baseline: 1.0281x; 1.0007x over previous
import functools

import jax
import jax.numpy as jnp
from jax import lax
from jax.experimental import pallas as pl
from jax.experimental.pallas import tpu as pltpu
from jax.experimental.pallas import tpu_sc as plsc

N_NODES = 10000
N_EDGES = 320000
D = 128
DH = D // 2
N_GRAPHS = 64
BN_EPS = 1e-5

NUM_CORES = 2
NUM_SUBCORES = 16
NUM_WORKERS = NUM_CORES * NUM_SUBCORES
E_PER_TILE = N_EDGES // NUM_WORKERS
CHUNK = 100
N_CHUNKS = E_PER_TILE // CHUNK
NBUF = 5
N_ITER = N_CHUNKS // NBUF
N_PAD = 10240
ROWS_PER_TILE = N_PAD // NUM_SUBCORES

_sc_mesh = plsc.VectorSubcoreMesh(
    core_axis_name="c", subcore_axis_name="s",
    num_cores=NUM_CORES, num_subcores=NUM_SUBCORES)


@functools.partial(
    pl.kernel,
    out_type=jax.ShapeDtypeStruct((NUM_CORES, 2, N_PAD, DH), jnp.float32),
    mesh=_sc_mesh,
    scratch_types=[
        pltpu.VMEM((N_CHUNKS, CHUNK), jnp.int32),
        pltpu.VMEM((N_CHUNKS, CHUNK), jnp.int32),
        pltpu.VMEM((CHUNK, DH), jnp.float32),
        pltpu.VMEM((CHUNK, DH), jnp.float32),
        pltpu.VMEM((CHUNK, DH), jnp.float32),
        pltpu.VMEM((CHUNK, DH), jnp.float32),
        pltpu.VMEM((CHUNK, DH), jnp.float32),
        pltpu.VMEM((ROWS_PER_TILE // 2, DH), jnp.float32),
        pltpu.VMEM_SHARED((N_PAD, DH), jnp.float32),
        pltpu.SemaphoreType.DMA,
        pltpu.SemaphoreType.DMA,
        pltpu.SemaphoreType.DMA,
        pltpu.SemaphoreType.DMA,
        pltpu.SemaphoreType.DMA,
    ],
    compiler_params=pltpu.CompilerParams(use_tc_tiling_on_sc=False, skip_device_barrier=True),
)
def _segment_sum_sc(hlo_hbm, hhi_hbm, col_hbm, row_hbm, out_hbm,
                    col_b, row_b, g0, g1, g2, g3, g4, zbuf, acc,
                    s0, s1, s2, s3, s4):
    bufs = (g0, g1, g2, g3, g4)
    sems = (s0, s1, s2, s3, s4)
    cid = lax.axis_index("c")
    sid = lax.axis_index("s")
    wid = sid * NUM_CORES + cid

    pltpu.sync_copy(col_hbm.at[wid], col_b)
    pltpu.sync_copy(row_hbm.at[wid], row_b)

    zv = jnp.zeros((16,), jnp.float32)

    def _zrow(r, carry):
        for c in range(DH // 16):
            zbuf[r, pl.ds(c * 16, 16)] = zv
        return carry

    lax.fori_loop(0, ROWS_PER_TILE // 2, _zrow, 0)

    for p, h_hbm in enumerate((hlo_hbm, hhi_hbm)):
        half = ROWS_PER_TILE // 2
        pltpu.sync_copy(zbuf, acc.at[pl.ds(sid * ROWS_PER_TILE, half)])
        pltpu.sync_copy(zbuf, acc.at[pl.ds(sid * ROWS_PER_TILE + half, half)])
        plsc.subcore_barrier()

        for b in range(NBUF):
            pltpu.async_copy(h_hbm.at[col_b.at[b]], bufs[b], sems[b])

        def _iter(i, carry):
            for b in range(NBUF):
                k = i * NBUF + b
                pltpu.make_async_copy(
                    h_hbm.at[col_b.at[0]], bufs[b], sems[b]).wait()
                pltpu.sync_copy(bufs[b], acc.at[row_b.at[k]], add=True)

                @pl.when(i < N_ITER - 1)
                def _fire():
                    pltpu.async_copy(
                        h_hbm.at[col_b.at[k + NBUF]], bufs[b], sems[b])
            return carry

        lax.fori_loop(0, N_ITER, _iter, 0)
        plsc.subcore_barrier()

        pltpu.sync_copy(
            acc.at[pl.ds(sid * ROWS_PER_TILE, ROWS_PER_TILE)],
            out_hbm.at[cid, p, pl.ds(sid * ROWS_PER_TILE, ROWS_PER_TILE)])


def _bn_relu(t, g, b):
    mean = jnp.mean(t, axis=0, keepdims=True)
    var = jnp.mean((t - mean) ** 2, axis=0, keepdims=True)
    return jnp.maximum(g * (t - mean) * lax.rsqrt(var + BN_EPS) + b, 0.0)


def _graph_pool(batch2d, h, pw, pb):
    gids = lax.broadcasted_iota(jnp.int32, (N_GRAPHS, N_NODES), 0)
    sel = (gids == batch2d).astype(jnp.float32)
    counts = jnp.maximum(jnp.sum(sel, axis=1, keepdims=True), 1.0)
    pooled = lax.dot_general(sel, h, (((1,), (0,)), ((), ())),
                             preferred_element_type=jnp.float32) / counts
    return lax.dot_general(pooled, pw, (((1,), (1,)), ((), ())),
                           preferred_element_type=jnp.float32) + pb


def _make_layer_tc(last):
    n_extra = 7 if last else 2

    def _layer_tc_body(eps_ref, part_ref, h_ref, w0_ref, b0_ref, g0_ref,
                       be0_ref, w1_ref, b1_ref, g1_ref, be1_ref, batch_ref,
                       pw_ref, pb_ref, *rest):
        if last:
            pw2_ref, pb2_ref, c0_ref, c1_ref, c2_ref, out_ref, score_ref = rest
        else:
            out_ref, score_ref = rest
        h = jnp.concatenate([h_ref[0], h_ref[1]], axis=1)
        seg = jnp.concatenate(
            [part_ref[0, 0, :N_NODES] + part_ref[1, 0, :N_NODES],
             part_ref[0, 1, :N_NODES] + part_ref[1, 1, :N_NODES]], axis=1)
        pooled = seg + (1.0 + eps_ref[0]) * h
        t = lax.dot_general(pooled, w0_ref[...], (((1,), (1,)), ((), ())),
                            preferred_element_type=jnp.float32) + b0_ref[...]
        t = _bn_relu(t, g0_ref[...], be0_ref[...])
        t = lax.dot_general(t, w1_ref[...], (((1,), (1,)), ((), ())),
                            preferred_element_type=jnp.float32) + b1_ref[...]
        y = _bn_relu(t, g1_ref[...], be1_ref[...])
        out_ref[0] = y[:, :DH]
        out_ref[1] = y[:, DH:]
        score = _graph_pool(batch_ref[...], h, pw_ref[...], pb_ref[...])
        if last:
            score = score + _graph_pool(batch_ref[...], y, pw2_ref[...],
                                        pb2_ref[...])
            score = score + c0_ref[...] + c1_ref[...] + c2_ref[...]
        score_ref[...] = score

    return pl.pallas_call(
        _layer_tc_body,
        out_shape=(jax.ShapeDtypeStruct((2, N_NODES, DH), jnp.float32),
                   jax.ShapeDtypeStruct((N_GRAPHS, D), jnp.float32)),
        in_specs=[pl.BlockSpec(memory_space=pltpu.SMEM)]
        + [pl.BlockSpec(memory_space=pltpu.VMEM)] * (11 + n_extra),
    )


_layer_tc = _make_layer_tc(False)
_layer_tc_last = _make_layer_tc(True)


def kernel(x, params, edge_index, batch):
    row = edge_index[0].reshape(NUM_WORKERS, N_CHUNKS, CHUNK)
    col = edge_index[1].reshape(NUM_WORKERS, N_CHUNKS, CHUNK)
    eps = params["eps"]
    h = jnp.stack([x[:, :DH], x[:, DH:]], axis=0)
    batch2d = batch.reshape(1, N_NODES)
    contribs = []
    for layer in range(4):
        mlp = params["mlp%d" % layer]
        parts = _segment_sum_sc(h[0], h[1], col, row)
        args = [
            eps[layer].reshape(1), parts, h,
            mlp["W0"], mlp["b0"].reshape(1, D),
            mlp["bn_g0"].reshape(1, D), mlp["bn_b0"].reshape(1, D),
            mlp["W1"], mlp["b1"].reshape(1, D),
            params["bn_g%d" % layer].reshape(1, D),
            params["bn_b%d" % layer].reshape(1, D),
            batch2d,
            params["pred%d_W" % layer], params["pred%d_b" % layer].reshape(1, D),
        ]
        if layer < 3:
            h, score = _layer_tc(*args)
            contribs.append(score)
        else:
            args += [params["pred4_W"], params["pred4_b"].reshape(1, D)]
            args += contribs
            h, score = _layer_tc_last(*args)
    return score
